# baseline (speedup 1.0000x reference)
import functools

import jax
import jax.numpy as jnp
from jax import lax
from jax.experimental import pallas as pl
from jax.experimental.pallas import tpu as pltpu
from jax.experimental.pallas import tpu_sc as plsc

NC = 2
NS = 16
NW = NC * NS
LANES = 16

BATCH = 4096
SEQ = 200
EMBED_DIM = 64

BPW = BATCH // NW
IDX_CHUNK = 100
CHUNKS_PER_ROW = SEQ // IDX_CHUNK
NV = EMBED_DIM // LANES
INV_SEQ = 1.0 / SEQ


def _pooled_sc(x2, table):
    mesh = plsc.VectorSubcoreMesh(core_axis_name="c", subcore_axis_name="s")

    @functools.partial(
        pl.kernel,
        mesh=mesh,
        compiler_params=pltpu.CompilerParams(use_tc_tiling_on_sc=False),
        out_type=jax.ShapeDtypeStruct((BATCH, EMBED_DIM), jnp.float32),
        scratch_types=[
            pltpu.VMEM((BPW * CHUNKS_PER_ROW, IDX_CHUNK), jnp.int32),
            pltpu.VMEM((SEQ, EMBED_DIM), jnp.float32),
            pltpu.VMEM((BPW, EMBED_DIM), jnp.float32),
            pltpu.SemaphoreType.DMA,
        ],
    )
    def k(x_hbm, table_hbm, out_hbm, idx_v, rows_v, pooled_v, sem):
        wid = lax.axis_index("s") * NC + lax.axis_index("c")
        base = wid * BPW
        pltpu.sync_copy(
            x_hbm.at[pl.ds(base * CHUNKS_PER_ROW, BPW * CHUNKS_PER_ROW)], idx_v
        )

        def row_body(i, _):
            copies = []
            for c in range(CHUNKS_PER_ROW):
                copies.append(
                    pltpu.async_copy(
                        table_hbm.at[idx_v.at[CHUNKS_PER_ROW * i + c]],
                        rows_v.at[pl.ds(c * IDX_CHUNK, IDX_CHUNK)],
                        sem,
                    )
                )
            for c in copies:
                c.wait()

            def red(r, accs):
                return tuple(
                    a + rows_v[r, pl.ds(j * LANES, LANES)]
                    for j, a in enumerate(accs)
                )

            accs = lax.fori_loop(
                0,
                SEQ,
                red,
                tuple(jnp.zeros((LANES,), jnp.float32) for _ in range(NV)),
            )
            for j in range(NV):
                pooled_v[i, pl.ds(j * LANES, LANES)] = accs[j] * INV_SEQ
            return _

        lax.fori_loop(0, BPW, row_body, None)
        pltpu.sync_copy(pooled_v, out_hbm.at[pl.ds(base, BPW)])

    return k(x2, table)


def _classifier_tc(pooled, W, b2):
    B, D = pooled.shape
    C = W.shape[0]
    BM = 512

    def mm(x_ref, w_ref, b_ref, o_ref):
        o_ref[...] = (
            lax.dot_general(
                x_ref[...],
                w_ref[...],
                (((1,), (1,)), ((), ())),
                preferred_element_type=jnp.float32,
            )
            + b_ref[...]
        )

    return pl.pallas_call(
        mm,
        grid=(B // BM,),
        in_specs=[
            pl.BlockSpec((BM, D), lambda i: (i, 0)),
            pl.BlockSpec((C, D), lambda i: (0, 0)),
            pl.BlockSpec((1, C), lambda i: (0, 0)),
        ],
        out_specs=pl.BlockSpec((BM, C), lambda i: (i, 0)),
        out_shape=jax.ShapeDtypeStruct((B, C), jnp.float32),
    )(pooled, W, b2)


def kernel(x_data, table, W, b):
    x2 = x_data.astype(jnp.int32).reshape(
        BATCH * CHUNKS_PER_ROW, IDX_CHUNK
    )
    pooled = _pooled_sc(x2, table)
    return _classifier_tc(pooled, W, b.reshape(1, -1))
